# baseline (speedup 1.0000x reference)
import functools

import jax
import jax.numpy as jnp
from jax import lax
from jax.experimental import pallas as pl
from jax.experimental.pallas import tpu as pltpu
from jax.experimental.pallas import tpu_sc as plsc

NUM_WORKERS = 32
CHUNK = 3200


def _body(x_hbm, table_hbm, out_hbm, xv, idx_v, dur_v, emb_v, out_v, sem,
          *, rows_per_worker, emb_dim, hidden):
    wid = lax.axis_index("s") * 2 + lax.axis_index("c")
    lanes = lax.broadcasted_iota(jnp.int32, (16,), 0)
    emb_mask = lanes < emb_dim
    num_chunks = rows_per_worker // CHUNK

    def chunk_body(k, carry):
        base = wid * rows_per_worker + k * CHUNK

        pltpu.sync_copy(x_hbm.at[pl.ds(2 * base, 2 * CHUNK)], xv)

        def split_body(j, carry):
            off = 32 * j + 2 * lanes
            fidx = plsc.load_gather(xv, [off])
            fdur = plsc.load_gather(xv, [off + 1])
            idx_v[pl.ds(16 * j, 16)] = fidx.astype(jnp.int32)
            dur_v[pl.ds(16 * j, 16)] = fdur
            return carry

        lax.fori_loop(0, CHUNK // 16, split_body, 0)

        pltpu.async_copy(table_hbm.at[idx_v], emb_v, sem).wait()

        def pack_body(r, carry):
            row = jnp.full((16,), r, jnp.int32)
            vec = plsc.load_gather(emb_v, [row, lanes], mask=emb_mask)
            out_v[pl.ds(hidden * r, 16)] = vec
            return carry

        lax.fori_loop(0, CHUNK, pack_body, 0)

        def dur_body(j, carry):
            dur = dur_v[pl.ds(16 * j, 16)]
            pos = hidden * (16 * j + lanes) + emb_dim
            plsc.store_scatter(out_v, [pos], dur)
            plsc.store_scatter(out_v, [pos + 1], dur)
            return carry

        lax.fori_loop(0, CHUNK // 16, dur_body, 0)

        pltpu.sync_copy(out_v, out_hbm.at[pl.ds(hidden * base, hidden * CHUNK)])
        return carry

    lax.fori_loop(0, num_chunks, chunk_body, 0)


def kernel(x, table):
    b, l, _ = x.shape
    vocab, emb_dim = table.shape
    hidden = emb_dim + 2
    n = b * l
    rows_per_worker = n // NUM_WORKERS

    x_flat = x.reshape(-1)

    body = functools.partial(
        _body,
        rows_per_worker=rows_per_worker,
        emb_dim=emb_dim,
        hidden=hidden,
    )

    run = pl.kernel(
        body,
        out_type=jax.ShapeDtypeStruct((n * hidden,), jnp.float32),
        mesh=plsc.VectorSubcoreMesh(core_axis_name="c", subcore_axis_name="s"),
        compiler_params=pltpu.CompilerParams(
            needs_layout_passes=False, use_tc_tiling_on_sc=False),
        scratch_types=[
            pltpu.VMEM((2 * CHUNK,), jnp.float32),
            pltpu.VMEM((CHUNK,), jnp.int32),
            pltpu.VMEM((CHUNK,), jnp.float32),
            pltpu.VMEM((CHUNK, emb_dim), jnp.float32),
            pltpu.VMEM((hidden * CHUNK,), jnp.float32),
            pltpu.SemaphoreType.DMA,
        ],
    )
    out_flat = run(x_flat, table)
    return out_flat.reshape(b, l, hidden)


# speedup vs baseline: 1.7730x; 1.7730x over previous
import jax
import jax.numpy as jnp
from jax import lax
from jax.experimental import pallas as pl
from jax.experimental.pallas import tpu as pltpu
from jax.experimental.pallas import tpu_sc as plsc

L_SEQ = 200
NB = 32
BI = 128
EMB = 14
LC = 25


def _body(tab_hbm, x_hbm, out_hbm, xchunk, idx_v, plane, col_a,
          sem_col, sem_g, sem_o):
    cid = lax.axis_index("c")
    sid = lax.axis_index("s")
    wid = sid * 2 + cid

    def stage_lc(lc, carry):
        l0 = LC * lc
        pltpu.sync_copy(x_hbm.at[pl.ds(l0, LC), wid, :, :], xchunk)
        pltpu.sync_copy(xchunk.at[:, 1, :], out_hbm.at[pl.ds(l0, LC), 1, wid, 6, :])
        pltpu.sync_copy(xchunk.at[:, 1, :], out_hbm.at[pl.ds(l0, LC), 1, wid, 7, :])

        def conv_l(l2, carry2):
            def conv_g(g, carry3):
                v = xchunk[l2, 0, pl.ds(16 * g, 16)]
                idx_v[l0 + l2, pl.ds(16 * g, 16)] = v.astype(jnp.int32)
                return carry3
            return lax.fori_loop(0, BI // 16, conv_g, carry2)

        return lax.fori_loop(0, LC, conv_l, carry)

    lax.fori_loop(0, L_SEQ // LC, stage_lc, 0)

    for c in range(EMB):
        @pl.when(sid == 0)
        def _():
            pltpu.async_copy(tab_hbm.at[c], col_a, sem_col).wait()

        plsc.subcore_barrier()

        if c >= 1:
            pltpu.make_async_copy(plane, out_hbm.at[:, 0, wid, 0, :], sem_o).wait()

        def fire_l(l, carry):
            pltpu.async_copy(col_a.at[idx_v.at[l]], plane.at[l], sem_g)
            return carry

        lax.fori_loop(0, L_SEQ, fire_l, 0)

        def drain_l(l, carry):
            pltpu.make_async_copy(col_a.at[idx_v.at[l]], plane.at[l], sem_g).wait()
            return carry

        lax.fori_loop(0, L_SEQ, drain_l, 0)

        pltpu.async_copy(plane, out_hbm.at[:, c // 8, wid, c % 8, :], sem_o)
        plsc.subcore_barrier()

    pltpu.make_async_copy(plane, out_hbm.at[:, 0, wid, 0, :], sem_o).wait()


def kernel(x, table):
    tab_t = table.T
    xp = x.transpose(1, 0, 2).reshape(L_SEQ, NB, BI, 2).transpose(0, 1, 3, 2)

    run = pl.kernel(
        _body,
        out_type=jax.ShapeDtypeStruct((L_SEQ, 2, NB, 8, BI), jnp.float32),
        mesh=plsc.VectorSubcoreMesh(core_axis_name="c", subcore_axis_name="s"),
        compiler_params=pltpu.CompilerParams(
            needs_layout_passes=False, use_tc_tiling_on_sc=False),
        scratch_types=[
            pltpu.VMEM((LC, 2, BI), jnp.float32),
            pltpu.VMEM((L_SEQ, BI), jnp.int32),
            pltpu.VMEM((L_SEQ, BI), jnp.float32),
            pltpu.VMEM_SHARED((1000000,), jnp.float32),
            pltpu.SemaphoreType.DMA,
            pltpu.SemaphoreType.DMA,
            pltpu.SemaphoreType.DMA,
        ],
    )
    out5 = run(tab_t, xp)
    out = out5.transpose(2, 4, 0, 1, 3).reshape(NB * BI, L_SEQ, EMB + 2)
    return out


# speedup vs baseline: 7.5768x; 4.2735x over previous
import jax
import jax.numpy as jnp
from jax import lax
from jax.experimental import pallas as pl
from jax.experimental.pallas import tpu as pltpu
from jax.experimental.pallas import tpu_sc as plsc

L_SEQ = 200
NB = 32
BI = 128
EMB = 14
LC = 25


def _body(tab_hbm, x_hbm, out_hbm, xchunk, idx_v, plane, col_a,
          sem_col, sem_g, sem_o):
    cid = lax.axis_index("c")
    sid = lax.axis_index("s")
    wid = sid * 2 + cid

    def stage_lc(lc, carry):
        l0 = LC * lc
        pltpu.sync_copy(x_hbm.at[pl.ds(l0, LC), wid, :, :], xchunk)
        pltpu.sync_copy(xchunk.at[:, 1, :], out_hbm.at[pl.ds(l0, LC), 1, wid, 6, :])
        pltpu.sync_copy(xchunk.at[:, 1, :], out_hbm.at[pl.ds(l0, LC), 1, wid, 7, :])

        def conv_l(l2, carry2):
            def conv_g(g, carry3):
                v = xchunk[l2, 0, pl.ds(16 * g, 16)]
                idx_v[l0 + l2, pl.ds(16 * g, 16)] = v.astype(jnp.int32)
                return carry3
            return lax.fori_loop(0, BI // 16, conv_g, carry2)

        return lax.fori_loop(0, LC, conv_l, carry)

    lax.fori_loop(0, L_SEQ // LC, stage_lc, 0)

    for c in range(EMB):
        @pl.when(sid == 0)
        def _():
            pltpu.async_copy(tab_hbm.at[c], col_a, sem_col).wait()

        plsc.subcore_barrier()

        if c >= 1:
            pltpu.make_async_copy(plane, out_hbm.at[:, 0, wid, 0, :], sem_o).wait()

        def fire_l(l, carry):
            pltpu.async_copy(col_a.at[idx_v.at[l]], plane.at[l], sem_g)
            return carry

        lax.fori_loop(0, L_SEQ, fire_l, 0)

        def drain_l(l, carry):
            pltpu.make_async_copy(col_a.at[idx_v.at[l]], plane.at[l], sem_g).wait()
            return carry

        lax.fori_loop(0, L_SEQ, drain_l, 0)

        pltpu.async_copy(plane, out_hbm.at[:, c // 8, wid, c % 8, :], sem_o)
        plsc.subcore_barrier()

    pltpu.make_async_copy(plane, out_hbm.at[:, 0, wid, 0, :], sem_o).wait()


VPAD = 1000448


def kernel(x, table):
    tab_p = jnp.pad(table, ((0, VPAD - 1000000), (0, 2)))
    tab_t = (tab_p.reshape(VPAD // BI, BI, 2, 8)
             .transpose(2, 0, 3, 1)
             .transpose(0, 2, 1, 3)
             .reshape(16, VPAD))
    xp = x.transpose(1, 0, 2).reshape(L_SEQ, NB, BI, 2).transpose(0, 1, 3, 2)

    run = pl.kernel(
        _body,
        out_type=jax.ShapeDtypeStruct((L_SEQ, 2, NB, 8, BI), jnp.float32),
        mesh=plsc.VectorSubcoreMesh(core_axis_name="c", subcore_axis_name="s"),
        compiler_params=pltpu.CompilerParams(
            needs_layout_passes=False, use_tc_tiling_on_sc=False),
        scratch_types=[
            pltpu.VMEM((LC, 2, BI), jnp.float32),
            pltpu.VMEM((L_SEQ, BI), jnp.int32),
            pltpu.VMEM((L_SEQ, BI), jnp.float32),
            pltpu.VMEM_SHARED((VPAD,), jnp.float32),
            pltpu.SemaphoreType.DMA,
            pltpu.SemaphoreType.DMA,
            pltpu.SemaphoreType.DMA,
        ],
    )
    out5 = run(tab_t, xp)
    out = out5.transpose(2, 4, 0, 1, 3).reshape(NB * BI, L_SEQ, EMB + 2)
    return out


# speedup vs baseline: 7.6024x; 1.0034x over previous
import jax
import jax.numpy as jnp
from jax import lax
from jax.experimental import pallas as pl
from jax.experimental.pallas import tpu as pltpu
from jax.experimental.pallas import tpu_sc as plsc

L_SEQ = 200
NB = 32
BI = 128
EMB = 14
LC = 25


def _body(tab_hbm, x_hbm, out_hbm, xchunk, idx_v, plane, col_a,
          sem_col, sem_g, sem_o):
    cid = lax.axis_index("c")
    sid = lax.axis_index("s")
    wid = sid * 2 + cid

    def stage_lc(lc, carry):
        l0 = LC * lc
        pltpu.sync_copy(x_hbm.at[pl.ds(l0, LC), wid, :, :], xchunk)
        pltpu.sync_copy(xchunk.at[:, 1, :], out_hbm.at[pl.ds(l0, LC), 1, wid, 6, :])
        pltpu.sync_copy(xchunk.at[:, 1, :], out_hbm.at[pl.ds(l0, LC), 1, wid, 7, :])

        def conv_l(l2, carry2):
            def conv_g(g, carry3):
                v = xchunk[l2, 0, pl.ds(16 * g, 16)]
                idx_v[l0 + l2, pl.ds(16 * g, 16)] = v.astype(jnp.int32)
                return carry3
            return lax.fori_loop(0, BI // 16, conv_g, carry2)

        return lax.fori_loop(0, LC, conv_l, carry)

    lax.fori_loop(0, L_SEQ // LC, stage_lc, 0)

    SLICE = VPAD // 16

    for c in range(EMB):
        pltpu.async_copy(tab_hbm.at[c, pl.ds(sid * SLICE, SLICE)],
                         col_a.at[pl.ds(sid * SLICE, SLICE)], sem_col).wait()
        plsc.subcore_barrier()

        if c >= 1:
            pltpu.make_async_copy(plane, out_hbm.at[:, 0, wid, 0, :], sem_o).wait()

        def fire_l(l, carry):
            pltpu.async_copy(col_a.at[idx_v.at[l]], plane.at[l], sem_g)
            return carry

        lax.fori_loop(0, L_SEQ, fire_l, 0)

        pltpu.make_async_copy(plane, out_hbm.at[:, 0, wid, 0, :], sem_g).wait()

        pltpu.async_copy(plane, out_hbm.at[:, c // 8, wid, c % 8, :], sem_o)
        plsc.subcore_barrier()

    pltpu.make_async_copy(plane, out_hbm.at[:, 0, wid, 0, :], sem_o).wait()


VPAD = 1000448


def kernel(x, table):
    tab_p = jnp.pad(table, ((0, VPAD - 1000000), (0, 2)))
    tab_t = (tab_p.reshape(VPAD // BI, BI, 2, 8)
             .transpose(2, 0, 3, 1)
             .transpose(0, 2, 1, 3)
             .reshape(16, VPAD))
    xp = x.transpose(1, 0, 2).reshape(L_SEQ, NB, BI, 2).transpose(0, 1, 3, 2)

    run = pl.kernel(
        _body,
        out_type=jax.ShapeDtypeStruct((L_SEQ, 2, NB, 8, BI), jnp.float32),
        mesh=plsc.VectorSubcoreMesh(core_axis_name="c", subcore_axis_name="s"),
        compiler_params=pltpu.CompilerParams(
            needs_layout_passes=False, use_tc_tiling_on_sc=False),
        scratch_types=[
            pltpu.VMEM((LC, 2, BI), jnp.float32),
            pltpu.VMEM((L_SEQ, BI), jnp.int32),
            pltpu.VMEM((L_SEQ, BI), jnp.float32),
            pltpu.VMEM_SHARED((VPAD,), jnp.float32),
            pltpu.SemaphoreType.DMA,
            pltpu.SemaphoreType.DMA,
            pltpu.SemaphoreType.DMA,
        ],
    )
    out5 = run(tab_t, xp)
    out = out5.transpose(2, 4, 0, 1, 3).reshape(NB * BI, L_SEQ, EMB + 2)
    return out
